# baseline (device time: 447124 ns/iter reference)
import jax
import jax.numpy as jnp
from jax import lax
from jax.experimental import pallas as pl
from jax.experimental.pallas import tpu as pltpu

M = 4096
N = 4096
KS = 2048
HM = M // 2
NC = 256
N_CHUNKS = N // NC
SLOTS = 4


def kernel(A, B):
    def body(
        a_hbm, b_hbm, out_hbm,
        a_buf, b_buf, pc_buf, px_buf, rc_buf, ry_buf,
        a_sem, b_sems, o_sems,
        x_send, x_recv, y_send, y_recv,
        x_credit, y_credit,
    ):
        my_x = lax.axis_index("x")
        my_y = lax.axis_index("y")
        x_nbr = (1 - my_x, my_y)
        y_nbr = (my_x, 1 - my_y)

        a_cp = pltpu.make_async_copy(
            a_hbm.at[pl.ds(my_y * HM, HM), :], a_buf, a_sem
        )
        a_cp.start()

        def b_load(c):
            return pltpu.make_async_copy(
                b_hbm.at[:, pl.ds(c * NC, NC)],
                b_buf.at[c % SLOTS],
                b_sems.at[c % SLOTS],
            )

        b_cps = {}
        for c in range(min(SLOTS, N_CHUNKS)):
            b_cps[c] = b_load(c)
            b_cps[c].start()

        barrier_sem = pltpu.get_barrier_semaphore()
        for nbr in (x_nbr, y_nbr):
            pl.semaphore_signal(
                barrier_sem, inc=1,
                device_id=nbr, device_id_type=pl.DeviceIdType.MESH,
            )
        pl.semaphore_wait(barrier_sem, 2)

        a_cp.wait()

        def x_desc(c):
            s = c % SLOTS
            return pltpu.make_async_remote_copy(
                src_ref=pc_buf.at[s], dst_ref=px_buf.at[s],
                send_sem=x_send.at[s], recv_sem=x_recv.at[s],
                device_id=x_nbr, device_id_type=pl.DeviceIdType.MESH,
            )

        def y_desc(c):
            s = c % SLOTS
            return pltpu.make_async_remote_copy(
                src_ref=rc_buf.at[s], dst_ref=ry_buf.at[s],
                send_sem=y_send.at[s], recv_sem=y_recv.at[s],
                device_id=y_nbr, device_id_type=pl.DeviceIdType.MESH,
            )

        x_rdmas = {}
        y_rdmas = {}
        for it in range(N_CHUNKS + 2):
            c = it
            if c < N_CHUNKS:
                s = c % SLOTS
                if c >= SLOTS:
                    x_rdmas[c - SLOTS].wait_send()
                b_cps[c].wait()
                pc_buf[s, :, :] = jnp.dot(
                    a_buf[:, :], b_buf[s], preferred_element_type=jnp.float32
                )
                if c + SLOTS < N_CHUNKS:
                    b_cps[c + SLOTS] = b_load(c + SLOTS)
                    b_cps[c + SLOTS].start()
                if c >= SLOTS:
                    pl.semaphore_wait(x_credit, 1)
                x_rdmas[c] = x_desc(c)
                x_rdmas[c].start()

            d = it - 1
            if 0 <= d < N_CHUNKS:
                sd = d % SLOTS
                x_rdmas[d].wait_recv()
                if d >= SLOTS:
                    y_rdmas[d - SLOTS].wait_send()
                rc_buf[sd, :, :] = pc_buf[sd] + px_buf[sd]
                if d + SLOTS < N_CHUNKS:
                    pl.semaphore_signal(
                        x_credit, inc=1,
                        device_id=x_nbr, device_id_type=pl.DeviceIdType.MESH,
                    )
                if d >= SLOTS:
                    pl.semaphore_wait(y_credit, 1)
                y_rdmas[d] = y_desc(d)
                y_rdmas[d].start()

            e = it - 2
            if 0 <= e < N_CHUNKS:
                se = e % SLOTS
                y_rdmas[e].wait_recv()
                o_cp0 = pltpu.make_async_copy(
                    rc_buf.at[se],
                    out_hbm.at[pl.ds(my_y * HM, HM), pl.ds(e * NC, NC)],
                    o_sems.at[0],
                )
                o_cp1 = pltpu.make_async_copy(
                    ry_buf.at[se],
                    out_hbm.at[pl.ds((1 - my_y) * HM, HM), pl.ds(e * NC, NC)],
                    o_sems.at[1],
                )
                o_cp0.start()
                o_cp1.start()
                o_cp0.wait()
                o_cp1.wait()
                if e + SLOTS < N_CHUNKS:
                    pl.semaphore_signal(
                        y_credit, inc=1,
                        device_id=y_nbr, device_id_type=pl.DeviceIdType.MESH,
                    )

        for c in range(max(0, N_CHUNKS - SLOTS), N_CHUNKS):
            x_rdmas[c].wait_send()
            y_rdmas[c].wait_send()

    return pl.pallas_call(
        body,
        out_shape=jax.ShapeDtypeStruct((M, N), jnp.float32),
        in_specs=[
            pl.BlockSpec(memory_space=pl.ANY),
            pl.BlockSpec(memory_space=pl.ANY),
        ],
        out_specs=pl.BlockSpec(memory_space=pl.ANY),
        scratch_shapes=[
            pltpu.VMEM((HM, KS), jnp.float32),
            pltpu.VMEM((SLOTS, KS, NC), jnp.float32),
            pltpu.VMEM((SLOTS, HM, NC), jnp.float32),
            pltpu.VMEM((SLOTS, HM, NC), jnp.float32),
            pltpu.VMEM((SLOTS, HM, NC), jnp.float32),
            pltpu.VMEM((SLOTS, HM, NC), jnp.float32),
            pltpu.SemaphoreType.DMA,
            pltpu.SemaphoreType.DMA((SLOTS,)),
            pltpu.SemaphoreType.DMA((2,)),
            pltpu.SemaphoreType.DMA((SLOTS,)),
            pltpu.SemaphoreType.DMA((SLOTS,)),
            pltpu.SemaphoreType.DMA((SLOTS,)),
            pltpu.SemaphoreType.DMA((SLOTS,)),
            pltpu.SemaphoreType.REGULAR,
            pltpu.SemaphoreType.REGULAR,
        ],
        compiler_params=pltpu.CompilerParams(
            collective_id=0,
            vmem_limit_bytes=63 * 1024 * 1024,
        ),
    )(A, B)


# device time: 256317 ns/iter; 1.7444x vs baseline; 1.7444x over previous
import jax
import jax.numpy as jnp
from jax import lax
from jax.experimental import pallas as pl
from jax.experimental.pallas import tpu as pltpu

M = 4096
N = 4096
KS = 2048
HM = M // 2
NC = 256
N_CHUNKS = N // NC
SLOTS = 3


def kernel(A, B):
    def body(
        a_hbm, b_hbm, out_hbm,
        af_buf, a16_buf, bf_buf, b16_buf,
        pc16_buf, px16_buf, rcf_buf, rc16_buf, ry16_buf, ost_buf,
        a_sem, b_sems, o_sems,
        x_send, x_recv, y_send, y_recv,
        x_credit, y_credit,
    ):
        my_x = lax.axis_index("x")
        my_y = lax.axis_index("y")
        x_nbr = (1 - my_x, my_y)
        y_nbr = (my_x, 1 - my_y)

        a_cp = pltpu.make_async_copy(
            a_hbm.at[pl.ds(my_y * HM, HM), :], af_buf, a_sem
        )
        a_cp.start()

        def b_load(c):
            return pltpu.make_async_copy(
                b_hbm.at[:, pl.ds(c * NC, NC)],
                bf_buf.at[c % SLOTS],
                b_sems.at[c % SLOTS],
            )

        b_cps = {}
        for c in range(min(SLOTS, N_CHUNKS)):
            b_cps[c] = b_load(c)
            b_cps[c].start()

        barrier_sem = pltpu.get_barrier_semaphore()
        for nbr in (x_nbr, y_nbr):
            pl.semaphore_signal(
                barrier_sem, inc=1,
                device_id=nbr, device_id_type=pl.DeviceIdType.MESH,
            )
        pl.semaphore_wait(barrier_sem, 2)

        a_cp.wait()
        a16_buf[:, :] = af_buf[:, :].astype(jnp.bfloat16)

        def x_desc(c):
            s = c % SLOTS
            return pltpu.make_async_remote_copy(
                src_ref=pc16_buf.at[s], dst_ref=px16_buf.at[s],
                send_sem=x_send.at[s], recv_sem=x_recv.at[s],
                device_id=x_nbr, device_id_type=pl.DeviceIdType.MESH,
            )

        def y_desc(c):
            s = c % SLOTS
            return pltpu.make_async_remote_copy(
                src_ref=rc16_buf.at[s], dst_ref=ry16_buf.at[s],
                send_sem=y_send.at[s], recv_sem=y_recv.at[s],
                device_id=y_nbr, device_id_type=pl.DeviceIdType.MESH,
            )

        x_rdmas = {}
        y_rdmas = {}
        for it in range(N_CHUNKS + 2):
            c = it
            if c < N_CHUNKS:
                s = c % SLOTS
                if c >= SLOTS:
                    x_rdmas[c - SLOTS].wait_send()
                b_cps[c].wait()
                b16_buf[s, :, :] = bf_buf[s].astype(jnp.bfloat16)
                pc16_buf[s, :, :] = jnp.dot(
                    a16_buf[:, :], b16_buf[s],
                    preferred_element_type=jnp.float32,
                ).astype(jnp.bfloat16)
                if c + SLOTS < N_CHUNKS:
                    b_cps[c + SLOTS] = b_load(c + SLOTS)
                    b_cps[c + SLOTS].start()
                if c >= SLOTS:
                    pl.semaphore_wait(x_credit, 1)
                x_rdmas[c] = x_desc(c)
                x_rdmas[c].start()

            d = it - 1
            if 0 <= d < N_CHUNKS:
                sd = d % SLOTS
                x_rdmas[d].wait_recv()
                if d >= SLOTS:
                    y_rdmas[d - SLOTS].wait_send()
                rsum = (
                    pc16_buf[sd].astype(jnp.float32)
                    + px16_buf[sd].astype(jnp.float32)
                )
                rcf_buf[d % 2, :, :] = rsum
                rc16_buf[sd, :, :] = rsum.astype(jnp.bfloat16)
                if d + SLOTS < N_CHUNKS:
                    pl.semaphore_signal(
                        x_credit, inc=1,
                        device_id=x_nbr, device_id_type=pl.DeviceIdType.MESH,
                    )
                if d >= SLOTS:
                    pl.semaphore_wait(y_credit, 1)
                y_rdmas[d] = y_desc(d)
                y_rdmas[d].start()

            e = it - 2
            if 0 <= e < N_CHUNKS:
                se = e % SLOTS
                y_rdmas[e].wait_recv()
                ost_buf[e % 2, :, :] = ry16_buf[se].astype(jnp.float32)
                o_cp0 = pltpu.make_async_copy(
                    rcf_buf.at[e % 2],
                    out_hbm.at[pl.ds(my_y * HM, HM), pl.ds(e * NC, NC)],
                    o_sems.at[0],
                )
                o_cp1 = pltpu.make_async_copy(
                    ost_buf.at[e % 2],
                    out_hbm.at[pl.ds((1 - my_y) * HM, HM), pl.ds(e * NC, NC)],
                    o_sems.at[1],
                )
                o_cp0.start()
                o_cp1.start()
                o_cp0.wait()
                o_cp1.wait()
                if e + SLOTS < N_CHUNKS:
                    pl.semaphore_signal(
                        y_credit, inc=1,
                        device_id=y_nbr, device_id_type=pl.DeviceIdType.MESH,
                    )

        for c in range(max(0, N_CHUNKS - SLOTS), N_CHUNKS):
            x_rdmas[c].wait_send()
            y_rdmas[c].wait_send()

    return pl.pallas_call(
        body,
        out_shape=jax.ShapeDtypeStruct((M, N), jnp.float32),
        in_specs=[
            pl.BlockSpec(memory_space=pl.ANY),
            pl.BlockSpec(memory_space=pl.ANY),
        ],
        out_specs=pl.BlockSpec(memory_space=pl.ANY),
        scratch_shapes=[
            pltpu.VMEM((HM, KS), jnp.float32),
            pltpu.VMEM((HM, KS), jnp.bfloat16),
            pltpu.VMEM((SLOTS, KS, NC), jnp.float32),
            pltpu.VMEM((SLOTS, KS, NC), jnp.bfloat16),
            pltpu.VMEM((SLOTS, HM, NC), jnp.bfloat16),
            pltpu.VMEM((SLOTS, HM, NC), jnp.bfloat16),
            pltpu.VMEM((2, HM, NC), jnp.float32),
            pltpu.VMEM((SLOTS, HM, NC), jnp.bfloat16),
            pltpu.VMEM((SLOTS, HM, NC), jnp.bfloat16),
            pltpu.VMEM((2, HM, NC), jnp.float32),
            pltpu.SemaphoreType.DMA,
            pltpu.SemaphoreType.DMA((SLOTS,)),
            pltpu.SemaphoreType.DMA((2,)),
            pltpu.SemaphoreType.DMA((SLOTS,)),
            pltpu.SemaphoreType.DMA((SLOTS,)),
            pltpu.SemaphoreType.DMA((SLOTS,)),
            pltpu.SemaphoreType.DMA((SLOTS,)),
            pltpu.SemaphoreType.REGULAR,
            pltpu.SemaphoreType.REGULAR,
        ],
        compiler_params=pltpu.CompilerParams(
            collective_id=0,
            vmem_limit_bytes=63 * 1024 * 1024,
        ),
    )(A, B)
